# baseline (speedup 1.0000x reference)
import jax
import jax.numpy as jnp
from jax import lax
from jax.experimental import pallas as pl
from jax.experimental.pallas import tpu as pltpu
from jax.experimental.pallas import tpu_sc as plsc

N_REQ = 4096
ROW = 32768
SUB = 8
LANE = 128
NC = 2
NS = 16
NW = NC * NS
B_PER_W = N_REQ // NW
L = 16


HALF = B_PER_W // 2


def _sc_body(table_hbm, pool_hbm, prefix_hbm, out_hbm, buf_v, sem, sem_a, sem_b):
    wid = lax.axis_index("s") * NC + lax.axis_index("c")
    base = wid * B_PER_W
    pool_v = buf_v.at[pl.ds(0, B_PER_W)]
    prefix_v = buf_v.at[pl.ds(B_PER_W, B_PER_W)]
    idx_v = buf_v.at[pl.ds(2 * B_PER_W, B_PER_W)]
    gath_v = buf_v.at[pl.ds(3 * B_PER_W, B_PER_W)]
    cp_pool = pltpu.async_copy(pool_hbm.at[pl.ds(base, B_PER_W)], pool_v, sem)
    cp_pref = pltpu.async_copy(prefix_hbm.at[pl.ds(base, B_PER_W)], prefix_v, sem)
    cp_pool.wait()
    cp_pref.wait()

    def compute_idx(i, _):
        s = pl.ds(i * L, L)
        p = pool_v[s]
        col = jnp.maximum(prefix_v[s] - 1, 0)
        idx_v[s] = (((p >> 3) * (ROW // LANE) + (col >> 7)) * (SUB * LANE)
                    + ((p & (SUB - 1)) << 7) + (col & (LANE - 1)))
        return 0

    lax.fori_loop(0, HALF // L, compute_idx, 0, unroll=2)
    g_a = pltpu.async_copy(
        table_hbm.at[buf_v.at[pl.ds(2 * B_PER_W, HALF)]],
        buf_v.at[pl.ds(3 * B_PER_W, HALF)], sem_a)
    lax.fori_loop(HALF // L, B_PER_W // L, compute_idx, 0, unroll=2)
    g_b = pltpu.async_copy(
        table_hbm.at[buf_v.at[pl.ds(2 * B_PER_W + HALF, HALF)]],
        buf_v.at[pl.ds(3 * B_PER_W + HALF, HALF)], sem_b)

    def apply_mask(i, _):
        s = pl.ds(i * L, L)
        gath_v[s] = jnp.where(prefix_v[s] > 0, gath_v[s], jnp.int32(-1))
        return 0

    g_a.wait()
    lax.fori_loop(0, HALF // L, apply_mask, 0, unroll=2)
    o_a = pltpu.async_copy(
        buf_v.at[pl.ds(3 * B_PER_W, HALF)],
        out_hbm.at[pl.ds(base, HALF)], sem_a)
    g_b.wait()
    lax.fori_loop(HALF // L, B_PER_W // L, apply_mask, 0, unroll=2)
    o_b = pltpu.async_copy(
        buf_v.at[pl.ds(3 * B_PER_W + HALF, HALF)],
        out_hbm.at[pl.ds(base + HALF, HALF)], sem_b)
    o_a.wait()
    o_b.wait()


@jax.jit
def _last_loc(table_flat, pool_idx, prefix_lens):
    mesh = plsc.VectorSubcoreMesh(core_axis_name="c", subcore_axis_name="s")
    return pl.kernel(
        _sc_body,
        mesh=mesh,
        out_type=jax.ShapeDtypeStruct((N_REQ,), jnp.int32),
        scratch_types=[
            pltpu.VMEM((4 * B_PER_W,), jnp.int32),
            pltpu.SemaphoreType.DMA,
            pltpu.SemaphoreType.DMA,
            pltpu.SemaphoreType.DMA,
        ],
        compiler_params=pltpu.CompilerParams(
            needs_layout_passes=False, skip_device_barrier=True),
    )(table_flat, pool_idx, prefix_lens)


def kernel(req_to_token, req_pool_indices_tensor, prefix_lens_tensor):
    r, c = req_to_token.shape
    table_flat = (
        req_to_token.reshape(r // SUB, SUB, c // LANE, LANE)
        .transpose(0, 2, 1, 3)
        .reshape(r * c)
    )
    return _last_loc(table_flat, req_pool_indices_tensor, prefix_lens_tensor)


# speedup vs baseline: 1.0111x; 1.0111x over previous
import jax
import jax.numpy as jnp
from jax import lax
from jax.experimental import pallas as pl
from jax.experimental.pallas import tpu as pltpu
from jax.experimental.pallas import tpu_sc as plsc

N_REQ = 4096
ROW = 32768
SUB = 8
LANE = 128
NC = 2
NS = 16
NW = NC * NS
B_PER_W = N_REQ // NW
L = 16


def _sc_body(table_hbm, pool_hbm, prefix_hbm, out_hbm, buf_v, sem):
    wid = lax.axis_index("s") * NC + lax.axis_index("c")
    base = wid * B_PER_W
    pool_v = buf_v.at[pl.ds(0, B_PER_W)]
    prefix_v = buf_v.at[pl.ds(B_PER_W, B_PER_W)]
    idx_v = buf_v.at[pl.ds(2 * B_PER_W, B_PER_W)]
    gath_v = buf_v.at[pl.ds(3 * B_PER_W, B_PER_W)]
    cp_pool = pltpu.async_copy(pool_hbm.at[pl.ds(base, B_PER_W)], pool_v, sem)
    cp_pref = pltpu.async_copy(prefix_hbm.at[pl.ds(base, B_PER_W)], prefix_v, sem)
    cp_pool.wait()
    cp_pref.wait()

    def compute_idx(i, _):
        s = pl.ds(i * L, L)
        p = pool_v[s]
        col = jnp.maximum(prefix_v[s] - 1, 0)
        idx_v[s] = (((p >> 3) * (ROW // LANE) + (col >> 7)) * (SUB * LANE)
                    + ((p & (SUB - 1)) << 7) + (col & (LANE - 1)))
        return 0

    lax.fori_loop(0, B_PER_W // L, compute_idx, 0, unroll=2)
    pltpu.async_copy(table_hbm.at[idx_v], gath_v, sem).wait()

    def apply_mask(i, _):
        s = pl.ds(i * L, L)
        gath_v[s] = jnp.where(prefix_v[s] > 0, gath_v[s], jnp.int32(-1))
        return 0

    lax.fori_loop(0, B_PER_W // L, apply_mask, 0, unroll=2)
    pltpu.sync_copy(gath_v, out_hbm.at[pl.ds(base, B_PER_W)])


@jax.jit
def _last_loc(table_flat, pool_idx, prefix_lens):
    mesh = plsc.VectorSubcoreMesh(core_axis_name="c", subcore_axis_name="s")
    return pl.kernel(
        _sc_body,
        mesh=mesh,
        out_type=jax.ShapeDtypeStruct((N_REQ,), jnp.int32),
        scratch_types=[
            pltpu.VMEM((4 * B_PER_W,), jnp.int32),
            pltpu.SemaphoreType.DMA,
        ],
        compiler_params=pltpu.CompilerParams(
            needs_layout_passes=False, skip_device_barrier=True),
    )(table_flat, pool_idx, prefix_lens)


def kernel(req_to_token, req_pool_indices_tensor, prefix_lens_tensor):
    r, c = req_to_token.shape
    table_flat = (
        req_to_token.reshape(r // SUB, SUB, c // LANE, LANE)
        .transpose(0, 2, 1, 3)
        .reshape(r * c)
    )
    return _last_loc(table_flat, req_pool_indices_tensor, prefix_lens_tensor)
